# Initial kernel scaffold; baseline (speedup 1.0000x reference)
import jax
import jax.numpy as jnp
from jax.experimental import pallas as pl


def kernel(x, pos, batch, x_skip, pos_skip, batch_skip, W, b):
    raise NotImplementedError("write your pallas kernel here")


# speedup vs baseline: 13.1342x; 13.1342x over previous
import jax
import jax.numpy as jnp
from jax.experimental import pallas as pl
from jax.experimental.pallas import tpu as pltpu

M = 16384
N = 4096
C = 128
CS = 64
BQ = 256
NBLK = M // BQ


def _fp_block(pos_skip_ref, x_skip_ref, posT_ref, x_ref, w1_ref, w2_ref,
              b_ref, out_ref, xw_s):
    @pl.when(pl.program_id(0) == 0)
    def _():
        xw_s[...] = jax.lax.dot_general(
            x_ref[...], w1_ref[...], (((1,), (1,)), ((), ())),
            preferred_element_type=jnp.float32)

    q = pos_skip_ref[...]
    p = posT_ref[...]
    qsq = jnp.sum(q * q, axis=1, keepdims=True)
    psq = jnp.sum(p * p, axis=0, keepdims=True)
    qb = q.astype(jnp.bfloat16).astype(jnp.float32)
    pb = p.astype(jnp.bfloat16).astype(jnp.float32)
    cross = (qb[:, 0:1] * pb[0:1, :]
             + qb[:, 1:2] * pb[1:2, :]
             + qb[:, 2:3] * pb[2:3, :])
    d2 = qsq + psq - 2.0 * cross

    lane = jax.lax.broadcasted_iota(jnp.int32, (BQ, N), 1)
    inf = jnp.float32(jnp.inf)
    sel_w = jnp.zeros((BQ, N), jnp.float32)
    wsum = jnp.zeros((BQ, 1), jnp.float32)
    for _ in range(3):
        m = jnp.min(d2, axis=1, keepdims=True)
        i = jnp.min(jnp.where(d2 == m, lane, N), axis=1,
                    keepdims=True)
        hit = lane == i
        w = 1.0 / jnp.maximum(m, 1e-16)
        sel_w = sel_w + jnp.where(hit, w, 0.0)
        wsum = wsum + w
        d2 = jnp.where(hit, inf, d2)

    acc = jnp.dot(sel_w, xw_s[...], preferred_element_type=jnp.float32)
    base = jax.lax.dot_general(
        x_skip_ref[...], w2_ref[...], (((1,), (1,)), ((), ())),
        preferred_element_type=jnp.float32)
    out_ref[...] = acc / wsum + base + b_ref[...]


def kernel(x, pos, batch, x_skip, pos_skip, batch_skip, W, b):
    w1 = W[:, :C]
    w2 = W[:, C:]
    posT = pos.T
    b2 = b.reshape(1, C)

    out = pl.pallas_call(
        _fp_block,
        grid=(NBLK,),
        in_specs=[
            pl.BlockSpec((BQ, 3), lambda i: (i, 0)),
            pl.BlockSpec((BQ, CS), lambda i: (i, 0)),
            pl.BlockSpec((3, N), lambda i: (0, 0)),
            pl.BlockSpec((N, C), lambda i: (0, 0)),
            pl.BlockSpec((C, C), lambda i: (0, 0)),
            pl.BlockSpec((C, CS), lambda i: (0, 0)),
            pl.BlockSpec((1, C), lambda i: (0, 0)),
        ],
        out_specs=pl.BlockSpec((BQ, C), lambda i: (i, 0)),
        out_shape=jax.ShapeDtypeStruct((M, C), jnp.float32),
        scratch_shapes=[pltpu.VMEM((N, C), jnp.float32)],
    )(pos_skip, x_skip, posT, x, w1, w2, b2)

    return (out, pos_skip, batch_skip)


# speedup vs baseline: 18.9791x; 1.4450x over previous
import jax
import jax.numpy as jnp
from jax.experimental import pallas as pl
from jax.experimental.pallas import tpu as pltpu

M = 16384
N = 4096
C = 128
CS = 64
BQ = 256
NBLK = M // BQ


def _fp_block(pos_skip_ref, x_skip_ref, posT_ref, x_ref, w1_ref, w2_ref,
              b_ref, out_ref, xw_s):
    @pl.when(pl.program_id(0) == 0)
    def _():
        xw_s[...] = jax.lax.dot_general(
            x_ref[...], w1_ref[...], (((1,), (1,)), ((), ())),
            preferred_element_type=jnp.float32)

    q = pos_skip_ref[...]
    p = posT_ref[...]
    qsq = jnp.sum(q * q, axis=1, keepdims=True)
    psq = jnp.sum(p * p, axis=0, keepdims=True)
    cross = jax.lax.dot_general(
        q, p, (((1,), (0,)), ((), ())),
        preferred_element_type=jnp.float32)
    d2 = (qsq + psq) - (cross + cross)

    inf = jnp.float32(jnp.inf)
    sel_w = jnp.zeros((BQ, N), jnp.float32)
    wsum = jnp.zeros((BQ, 1), jnp.float32)
    for _ in range(3):
        m = jnp.min(d2, axis=1, keepdims=True)
        hit = d2 == m
        w = 1.0 / jnp.maximum(m, 1e-16)
        sel_w = sel_w + jnp.where(hit, w, 0.0)
        wsum = wsum + w
        d2 = jnp.where(hit, inf, d2)

    acc = jnp.dot(sel_w, xw_s[...], preferred_element_type=jnp.float32)
    base = jax.lax.dot_general(
        x_skip_ref[...], w2_ref[...], (((1,), (1,)), ((), ())),
        preferred_element_type=jnp.float32)
    out_ref[...] = acc / wsum + base + b_ref[...]


def kernel(x, pos, batch, x_skip, pos_skip, batch_skip, W, b):
    w1 = W[:, :C]
    w2 = W[:, C:]
    posT = pos.T.astype(jnp.bfloat16).astype(jnp.float32)
    ps_r = pos_skip.astype(jnp.bfloat16).astype(jnp.float32)
    b2 = b.reshape(1, C)

    out = pl.pallas_call(
        _fp_block,
        grid=(NBLK,),
        in_specs=[
            pl.BlockSpec((BQ, 3), lambda i: (i, 0)),
            pl.BlockSpec((BQ, CS), lambda i: (i, 0)),
            pl.BlockSpec((3, N), lambda i: (0, 0)),
            pl.BlockSpec((N, C), lambda i: (0, 0)),
            pl.BlockSpec((C, C), lambda i: (0, 0)),
            pl.BlockSpec((C, CS), lambda i: (0, 0)),
            pl.BlockSpec((1, C), lambda i: (0, 0)),
        ],
        out_specs=pl.BlockSpec((BQ, C), lambda i: (i, 0)),
        out_shape=jax.ShapeDtypeStruct((M, C), jnp.float32),
        scratch_shapes=[pltpu.VMEM((N, C), jnp.float32)],
    )(ps_r, x_skip, posT, x, w1, w2, b2)

    return (out, pos_skip, batch_skip)


# speedup vs baseline: 20.1885x; 1.0637x over previous
import jax
import jax.numpy as jnp
from jax.experimental import pallas as pl
from jax.experimental.pallas import tpu as pltpu

M = 16384
N = 4096
C = 128
CS = 64
BQ = 256
NBLK = M // BQ


def _fp_block(pos_skip_ref, x_skip_ref, posT_ref, x_ref, w1_ref, w2_ref,
              b_ref, out_ref, xw_s):
    @pl.when(pl.program_id(0) == 0)
    def _():
        xw_s[...] = jax.lax.dot_general(
            x_ref[...], w1_ref[...], (((1,), (1,)), ((), ())),
            preferred_element_type=jnp.float32)

    q = pos_skip_ref[...]
    p = posT_ref[...]
    qsq = jnp.sum(q * q, axis=1, keepdims=True)
    psq = jnp.sum(p * p, axis=0, keepdims=True)
    cross = jax.lax.dot_general(
        q, p, (((1,), (0,)), ((), ())),
        preferred_element_type=jnp.float32)
    e = psq - (cross + cross)

    inf = jnp.float32(jnp.inf)
    sel_w = jnp.zeros((BQ, N), jnp.float32)
    wsum = jnp.zeros((BQ, 1), jnp.float32)
    for _ in range(3):
        m = jnp.min(e, axis=1, keepdims=True)
        hit = e == m
        w = 1.0 / jnp.maximum(m + qsq, 1e-16)
        sel_w = jnp.where(hit, w, sel_w)
        wsum = wsum + w
        e = jnp.where(hit, inf, e)

    acc = jnp.dot(sel_w, xw_s[...], preferred_element_type=jnp.float32)
    base = jax.lax.dot_general(
        x_skip_ref[...], w2_ref[...], (((1,), (1,)), ((), ())),
        preferred_element_type=jnp.float32)
    out_ref[...] = acc / wsum + base + b_ref[...]


def kernel(x, pos, batch, x_skip, pos_skip, batch_skip, W, b):
    w1 = W[:, :C]
    w2 = W[:, C:]
    posT = pos.T.astype(jnp.bfloat16).astype(jnp.float32)
    ps_r = pos_skip.astype(jnp.bfloat16).astype(jnp.float32)
    b2 = b.reshape(1, C)

    out = pl.pallas_call(
        _fp_block,
        grid=(NBLK,),
        in_specs=[
            pl.BlockSpec((BQ, 3), lambda i: (i, 0)),
            pl.BlockSpec((BQ, CS), lambda i: (i, 0)),
            pl.BlockSpec((3, N), lambda i: (0, 0)),
            pl.BlockSpec((N, C), lambda i: (0, 0)),
            pl.BlockSpec((C, C), lambda i: (0, 0)),
            pl.BlockSpec((C, CS), lambda i: (0, 0)),
            pl.BlockSpec((1, C), lambda i: (0, 0)),
        ],
        out_specs=pl.BlockSpec((BQ, C), lambda i: (i, 0)),
        out_shape=jax.ShapeDtypeStruct((M, C), jnp.float32),
        scratch_shapes=[pltpu.VMEM((N, C), jnp.float32)],
    )(ps_r, x_skip, posT, x, w1, w2, b2)

    return (out, pos_skip, batch_skip)
